# Initial kernel scaffold; baseline (speedup 1.0000x reference)
import jax
import jax.numpy as jnp
from jax.experimental import pallas as pl


def kernel(x, edge_index, W1, b1, W3, b3):
    raise NotImplementedError("write your pallas kernel here")


# speedup vs baseline: 9.4879x; 9.4879x over previous
import functools

import jax
import jax.numpy as jnp
from jax import lax
from jax.experimental import pallas as pl
from jax.experimental.pallas import tpu as pltpu
from jax.experimental.pallas import tpu_sc as plsc

N_NODES = 10000
N_EDGES = 160000
F_IN = 256
F_HID = 128

NCORE = 2
NSUB = 16
NW = NCORE * NSUB
CHUNK = 128
EPW = 5120
NCHUNK = EPW // CHUNK
E_PAD = EPW * NW
N_ACC = 10112
ROWS_ACC = N_ACC // NSUB
ROW_BLK = 1000


def _vmesh():
    return plsc.VectorSubcoreMesh(core_axis_name="c", subcore_axis_name="s")


def _sc_deg(dstp, ones128, zeros128):
    @functools.partial(
        pl.kernel,
        out_type=jax.ShapeDtypeStruct((NCORE, N_ACC, F_HID), jnp.float32),
        mesh=_vmesh(),
        scratch_types=[
            pltpu.VMEM((NCHUNK, CHUNK), jnp.int32),
            pltpu.VMEM((CHUNK, F_HID), jnp.float32),
            pltpu.VMEM_SHARED((N_ACC, F_HID), jnp.float32),
        ],
    )
    def k(dst_hbm, ones_hbm, zeros_hbm, deg_hbm, dst_v, ones_v, acc_sh):
        c = lax.axis_index("c")
        s = lax.axis_index("s")
        w = c * NSUB + s
        pltpu.sync_copy(dst_hbm.at[w], dst_v)
        pltpu.sync_copy(ones_hbm, ones_v)
        pltpu.sync_copy(zeros_hbm.at[pl.ds(s * ROWS_ACC, ROWS_ACC)],
                        acc_sh.at[pl.ds(s * ROWS_ACC, ROWS_ACC)])
        plsc.subcore_barrier()

        @pl.loop(0, NCHUNK)
        def _(j):
            pltpu.sync_copy(ones_v, acc_sh.at[dst_v.at[j]], add=True)

        plsc.subcore_barrier()
        pltpu.sync_copy(acc_sh.at[pl.ds(s * ROWS_ACC, ROWS_ACC)],
                        deg_hbm.at[c, pl.ds(s * ROWS_ACC, ROWS_ACC)])

    return k(dstp, ones128, zeros128)


def _sc_scatter(z, srcp, dstp, zeros128):
    @functools.partial(
        pl.kernel,
        out_type=jax.ShapeDtypeStruct((NCORE, N_ACC, F_HID), jnp.float32),
        mesh=_vmesh(),
        scratch_types=[
            pltpu.VMEM((NCHUNK, CHUNK), jnp.int32),
            pltpu.VMEM((NCHUNK, CHUNK), jnp.int32),
            pltpu.VMEM((2, CHUNK, F_HID), jnp.float32),
            pltpu.VMEM_SHARED((N_ACC, F_HID), jnp.float32),
            pltpu.SemaphoreType.DMA,
            pltpu.SemaphoreType.DMA,
        ],
    )
    def k(z_hbm, src_hbm, dst_hbm, zeros_hbm, out_hbm,
          src_v, dst_v, buf, acc_sh, sem0, sem1):
        c = lax.axis_index("c")
        s = lax.axis_index("s")
        w = c * NSUB + s
        pltpu.sync_copy(src_hbm.at[w], src_v)
        pltpu.sync_copy(dst_hbm.at[w], dst_v)
        pltpu.sync_copy(zeros_hbm.at[pl.ds(s * ROWS_ACC, ROWS_ACC)],
                        acc_sh.at[pl.ds(s * ROWS_ACC, ROWS_ACC)])
        plsc.subcore_barrier()

        pltpu.async_copy(z_hbm.at[src_v.at[0]], buf.at[0], sem0)
        pltpu.async_copy(z_hbm.at[src_v.at[1]], buf.at[1], sem1)

        @pl.loop(0, NCHUNK, step=2)
        def _(j):
            pltpu.make_async_copy(z_hbm.at[src_v.at[j]], buf.at[0], sem0).wait()
            pltpu.sync_copy(buf.at[0], acc_sh.at[dst_v.at[j]], add=True)

            @pl.when(j + 2 < NCHUNK)
            def _():
                pltpu.async_copy(z_hbm.at[src_v.at[j + 2]], buf.at[0], sem0)

            pltpu.make_async_copy(z_hbm.at[src_v.at[j + 1]], buf.at[1], sem1).wait()
            pltpu.sync_copy(buf.at[1], acc_sh.at[dst_v.at[j + 1]], add=True)

            @pl.when(j + 3 < NCHUNK)
            def _():
                pltpu.async_copy(z_hbm.at[src_v.at[j + 3]], buf.at[1], sem1)

        plsc.subcore_barrier()
        pltpu.sync_copy(acc_sh.at[pl.ds(s * ROWS_ACC, ROWS_ACC)],
                        out_hbm.at[c, pl.ds(s * ROWS_ACC, ROWS_ACC)])

    return k(z, srcp, dstp, zeros128)


def _dinv_block(d_ref):
    d = d_ref[0][:, 0:1] + d_ref[1][:, 0:1] + 1.0
    return lax.rsqrt(d)


def _tc_lin1(x, W1, deg):
    def body(x_ref, w_ref, d_ref, o_ref):
        y = jnp.dot(x_ref[...], w_ref[...], preferred_element_type=jnp.float32)
        o_ref[...] = y * _dinv_block(d_ref)

    return pl.pallas_call(
        body,
        grid=(N_NODES // ROW_BLK,),
        in_specs=[
            pl.BlockSpec((ROW_BLK, F_IN), lambda i: (i, 0)),
            pl.BlockSpec((F_IN, F_HID), lambda i: (0, 0)),
            pl.BlockSpec((NCORE, ROW_BLK, F_HID), lambda i: (0, i, 0)),
        ],
        out_specs=pl.BlockSpec((ROW_BLK, F_HID), lambda i: (i, 0)),
        out_shape=jax.ShapeDtypeStruct((N_NODES, F_HID), jnp.float32),
    )(x, W1, deg)


def _tc_lin2(acc, z1, deg, b1, W3):
    def body(a_ref, z_ref, d_ref, b_ref, w_ref, o_ref):
        dinv = _dinv_block(d_ref)
        h = (a_ref[0] + a_ref[1] + z_ref[...]) * dinv + b_ref[...]
        h = jnp.maximum(h, 0.0)
        y = jnp.dot(h, w_ref[...], preferred_element_type=jnp.float32)
        o_ref[...] = y * dinv

    return pl.pallas_call(
        body,
        grid=(N_NODES // ROW_BLK,),
        in_specs=[
            pl.BlockSpec((NCORE, ROW_BLK, F_HID), lambda i: (0, i, 0)),
            pl.BlockSpec((ROW_BLK, F_HID), lambda i: (i, 0)),
            pl.BlockSpec((NCORE, ROW_BLK, F_HID), lambda i: (0, i, 0)),
            pl.BlockSpec((1, F_HID), lambda i: (0, 0)),
            pl.BlockSpec((F_HID, F_HID), lambda i: (0, 0)),
        ],
        out_specs=pl.BlockSpec((ROW_BLK, F_HID), lambda i: (i, 0)),
        out_shape=jax.ShapeDtypeStruct((N_NODES, F_HID), jnp.float32),
    )(acc, z1, deg, b1, W3)


def _tc_final(acc, z2, deg, b3):
    def body(a_ref, z_ref, d_ref, b_ref, o_ref):
        dinv = _dinv_block(d_ref)
        o_ref[...] = (a_ref[0] + a_ref[1] + z_ref[...]) * dinv + b_ref[...]

    return pl.pallas_call(
        body,
        grid=(N_NODES // ROW_BLK,),
        in_specs=[
            pl.BlockSpec((NCORE, ROW_BLK, F_HID), lambda i: (0, i, 0)),
            pl.BlockSpec((ROW_BLK, F_HID), lambda i: (i, 0)),
            pl.BlockSpec((NCORE, ROW_BLK, F_HID), lambda i: (0, i, 0)),
            pl.BlockSpec((1, F_HID), lambda i: (0, 0)),
        ],
        out_specs=pl.BlockSpec((ROW_BLK, F_HID), lambda i: (i, 0)),
        out_shape=jax.ShapeDtypeStruct((N_NODES, F_HID), jnp.float32),
    )(acc, z2, deg, b3)


def kernel(x, edge_index, W1, b1, W3, b3):
    src = edge_index[0]
    dst = edge_index[1]
    pad_src = jnp.zeros((E_PAD - N_EDGES,), jnp.int32)
    pad_dst = jnp.full((E_PAD - N_EDGES,), N_NODES, jnp.int32)
    srcp = jnp.concatenate([src, pad_src]).reshape(NW, NCHUNK, CHUNK)
    dstp = jnp.concatenate([dst, pad_dst]).reshape(NW, NCHUNK, CHUNK)
    ones128 = jnp.ones((CHUNK, F_HID), jnp.float32)
    zeros128 = jnp.zeros((N_ACC, F_HID), jnp.float32)

    deg = _sc_deg(dstp, ones128, zeros128)
    z1 = _tc_lin1(x, W1, deg)
    acc1 = _sc_scatter(z1, srcp, dstp, zeros128)
    z2 = _tc_lin2(acc1, z1, deg, b1.reshape(1, F_HID), W3)
    acc2 = _sc_scatter(z2, srcp, dstp, zeros128)
    return _tc_final(acc2, z2, deg, b3.reshape(1, F_HID))
